# baseline (speedup 1.0000x reference)
import jax
import jax.numpy as jnp
from jax.experimental import pallas as pl
from jax.experimental.pallas import tpu as pltpu

B = 32
T = 4
H = 512
E = 8


def _gates(g):
    i = jax.nn.sigmoid(g[:, :H])
    f = jax.nn.sigmoid(g[:, H:2 * H])
    gg = jnp.tanh(g[:, 2 * H:3 * H])
    o = jax.nn.sigmoid(g[:, 3 * H:])
    return i, f, gg, o


def _cell(gx, gh, b, c):
    i, f, gg, o = _gates(gx + gh + b)
    cn = f * c + i * gg
    hn = o * jnp.tanh(cn)
    return hn, cn


def _cell0(gx, b):
    i, _, gg, o = _gates(gx + b)
    cn = i * gg
    hn = o * jnp.tanh(cn)
    return hn, cn


def _dot(a, b):
    return jax.lax.dot_general(a, b, (((1,), (0,)), ((), ())),
                               preferred_element_type=jnp.float32)


def _moe_kernel(x_ref, d_b1_ref, d_b2_ref, g_W_ref, g_b_ref,
                e_b1_ref, e_b2_ref, e_bo_ref,
                d_Wi1_v, d_Wi2_v, e_Wi1_v, e_Wi2_v,
                e_Wh1_v, e_Wh2_v, e_Wo_v,
                d_Wh1_h, d_Wh2_h,
                out_ref,
                x0_s,
                d_Wh1, d_Wh2,
                *sems):
    hbm = (d_Wh1_h, d_Wh2_h)
    vmem = (d_Wh1, d_Wh2)
    copies = {}
    cx = pltpu.make_async_copy(x_ref.at[:, 0, :], x0_s, sems[-1])
    cx.start()
    for src, dst, sem in zip(hbm, vmem, sems):
        c = pltpu.make_async_copy(src, dst, sem)
        c.start()
        copies[id(dst)] = c

    waited = set()

    def w(ref):
        if id(ref) not in waited:
            copies[id(ref)].wait()
            waited.add(id(ref))
        return ref[...]

    cx.wait()
    x0 = x0_s[...]
    d_b1 = d_b1_ref[...]
    d_b2 = d_b2_ref[...]
    e_b1 = e_b1_ref[...]
    e_b2 = e_b2_ref[...]
    e_bo = e_bo_ref[...]
    g_b = g_b_ref[...]

    lane = jax.lax.broadcasted_iota(jnp.int32, (B, E), 1)

    def gate_weights(d_c2):
        logits = _dot(d_c2, g_W_ref[...]) + g_b
        m = jnp.max(logits, axis=1, keepdims=True)
        ex = jnp.exp(logits - m)
        p = ex / jnp.sum(ex, axis=1, keepdims=True)
        m1 = jnp.max(p, axis=1, keepdims=True)
        i1 = jnp.min(jnp.where(p == m1, lane, E), axis=1, keepdims=True)
        p2 = jnp.where(lane == i1, -1.0, p)
        m2 = jnp.max(p2, axis=1, keepdims=True)
        i2 = jnp.min(jnp.where(p2 == m2, lane, E), axis=1, keepdims=True)
        w_lo = jnp.where(i1 < i2, m1, m2)
        w_hi = jnp.where(i1 < i2, m2, m1)
        return w_lo, w_hi

    d_h1, d_c1 = _cell0(_dot(x0, d_Wi1_v[...]), d_b1)
    d_h2, d_c2 = _cell0(_dot(d_h1, d_Wi2_v[...]), d_b2)
    w_lo, w_hi = gate_weights(d_c2)

    xw = _dot(x0, e_Wi1_v[...])
    h1a, c1a = _cell0(xw, e_b1)
    h2a, c2a = _cell0(_dot(h1a, e_Wi2_v[...]), e_b2)
    out_a = _dot(h2a, e_Wo_v[...]) + e_bo
    h1b, c1b = _cell(xw, _dot(h1a, e_Wh1_v[...]), e_b1, c1a)
    h2b, c2b = _cell(_dot(h1b, e_Wi2_v[...]), _dot(h2a, e_Wh2_v[...]),
                     e_b2, c2a)
    out_b = _dot(h2b, e_Wo_v[...]) + e_bo
    e_h1, e_c1, e_h2, e_c2 = h1b, c1b, h2b, c2b

    o = w_lo * out_a + w_hi * out_b
    out_ref[:, 0, :] = o

    for t in range(1, T):
        xt = o
        d_h1, d_c1 = _cell(_dot(xt, d_Wi1_v[...]),
                           _dot(d_h1, w(d_Wh1)), d_b1, d_c1)
        d_h2, d_c2 = _cell(_dot(d_h1, d_Wi2_v[...]),
                           _dot(d_h2, w(d_Wh2)), d_b2, d_c2)
        w_lo, w_hi = gate_weights(d_c2)

        xw = _dot(xt, e_Wi1_v[...])
        h1a, c1a = _cell(xw, _dot(e_h1, e_Wh1_v[...]), e_b1, e_c1)
        h2a, c2a = _cell(_dot(h1a, e_Wi2_v[...]),
                         _dot(e_h2, e_Wh2_v[...]), e_b2, e_c2)
        out_a = _dot(h2a, e_Wo_v[...]) + e_bo
        h1b, c1b = _cell(xw, _dot(h1a, e_Wh1_v[...]), e_b1, c1a)
        h2b, c2b = _cell(_dot(h1b, e_Wi2_v[...]),
                         _dot(h2a, e_Wh2_v[...]), e_b2, c2a)
        out_b = _dot(h2b, e_Wo_v[...]) + e_bo
        e_h1, e_c1, e_h2, e_c2 = h1b, c1b, h2b, c2b

        o = w_lo * out_a + w_hi * out_b
        out_ref[:, t, :] = o


def kernel(x, d_Wi1, d_Wh1, d_b1, d_Wi2, d_Wh2, d_b2, d_Wo, d_bo,
           g_W, g_b,
           e_Wi1, e_Wh1, e_b1, e_Wi2, e_Wh2, e_b2, e_Wo, e_bo):
    del d_Wo, d_bo
    n_small = 8
    n_big = 2
    big_shapes = [(512, 2048)] * 2
    out = pl.pallas_call(
        _moe_kernel,
        out_shape=jax.ShapeDtypeStruct((B, T, H), jnp.float32),
        in_specs=(
            [pl.BlockSpec(memory_space=pltpu.MemorySpace.HBM)]
            + [pl.BlockSpec(memory_space=pltpu.MemorySpace.VMEM)] * (n_small - 1)
            + [pl.BlockSpec(memory_space=pltpu.MemorySpace.VMEM)] * 7
            + [pl.BlockSpec(memory_space=pltpu.MemorySpace.HBM)] * n_big),
        out_specs=pl.BlockSpec(memory_space=pltpu.MemorySpace.VMEM),
        scratch_shapes=(
            [pltpu.VMEM((B, H), jnp.float32)]
            + [pltpu.VMEM(s, jnp.float32) for s in big_shapes]
            + [pltpu.SemaphoreType.DMA] * (n_big + 1)),
    )(x, d_b1.reshape(1, -1), d_b2.reshape(1, -1), g_W, g_b.reshape(1, -1),
      e_b1.reshape(1, -1), e_b2.reshape(1, -1), e_bo.reshape(1, -1),
      d_Wi1, d_Wi2, e_Wi1, e_Wi2, e_Wh1, e_Wh2, e_Wo,
      d_Wh1, d_Wh2)
    return out


# speedup vs baseline: 1.0534x; 1.0534x over previous
import jax
import jax.numpy as jnp
from jax.experimental import pallas as pl
from jax.experimental.pallas import tpu as pltpu

B = 32
T = 4
H = 512
E = 8


def _gates(g):
    i = jax.nn.sigmoid(g[:, :H])
    f = jax.nn.sigmoid(g[:, H:2 * H])
    gg = jnp.tanh(g[:, 2 * H:3 * H])
    o = jax.nn.sigmoid(g[:, 3 * H:])
    return i, f, gg, o


def _cell(gx, gh, b, c):
    i, f, gg, o = _gates(gx + gh + b)
    cn = f * c + i * gg
    hn = o * jnp.tanh(cn)
    return hn, cn


def _cell0(gx, b):
    i, _, gg, o = _gates(gx + b)
    cn = i * gg
    hn = o * jnp.tanh(cn)
    return hn, cn


def _dot(a, b):
    return jax.lax.dot_general(a, b, (((1,), (0,)), ((), ())),
                               preferred_element_type=jnp.float32)


def _moe_kernel(x_ref, d_b1_ref, d_b2_ref, g_W_ref, g_b_ref,
                e_b1_ref, e_b2_ref, e_bo_ref,
                d_Wi1_v, d_Wi2_v, e_Wi1_v, e_Wi2_v,
                d_Wh1_h, d_Wh2_h, e_Wh1_h, e_Wh2_h, e_Wo_h,
                out_ref,
                x0_s,
                d_Wh1, d_Wh2, e_Wh1, e_Wh2, e_Wo,
                *sems):
    hbm = (e_Wo_h, e_Wh1_h, e_Wh2_h, d_Wh1_h, d_Wh2_h)
    vmem = (e_Wo, e_Wh1, e_Wh2, d_Wh1, d_Wh2)
    copies = {}
    cx = pltpu.make_async_copy(x_ref.at[:, 0, :], x0_s, sems[-1])
    cx.start()
    for src, dst, sem in zip(hbm, vmem, sems):
        c = pltpu.make_async_copy(src, dst, sem)
        c.start()
        copies[id(dst)] = c

    waited = set()

    def w(ref):
        if id(ref) not in waited:
            copies[id(ref)].wait()
            waited.add(id(ref))
        return ref[...]

    cx.wait()
    x0 = x0_s[...]
    d_b1 = d_b1_ref[...]
    d_b2 = d_b2_ref[...]
    e_b1 = e_b1_ref[...]
    e_b2 = e_b2_ref[...]
    e_bo = e_bo_ref[...]
    g_b = g_b_ref[...]

    lane = jax.lax.broadcasted_iota(jnp.int32, (B, E), 1)

    def gate_weights(d_c2):
        logits = _dot(d_c2, g_W_ref[...]) + g_b
        m = jnp.max(logits, axis=1, keepdims=True)
        ex = jnp.exp(logits - m)
        p = ex / jnp.sum(ex, axis=1, keepdims=True)
        m1 = jnp.max(p, axis=1, keepdims=True)
        i1 = jnp.min(jnp.where(p == m1, lane, E), axis=1, keepdims=True)
        p2 = jnp.where(lane == i1, -1.0, p)
        m2 = jnp.max(p2, axis=1, keepdims=True)
        i2 = jnp.min(jnp.where(p2 == m2, lane, E), axis=1, keepdims=True)
        w_lo = jnp.where(i1 < i2, m1, m2)
        w_hi = jnp.where(i1 < i2, m2, m1)
        return w_lo, w_hi

    d_h1, d_c1 = _cell0(_dot(x0, d_Wi1_v[...]), d_b1)
    d_h2, d_c2 = _cell0(_dot(d_h1, d_Wi2_v[...]), d_b2)
    w_lo, w_hi = gate_weights(d_c2)

    xw = _dot(x0, e_Wi1_v[...])
    h1a, c1a = _cell0(xw, e_b1)
    h2a, c2a = _cell0(_dot(h1a, e_Wi2_v[...]), e_b2)
    out_a = _dot(h2a, w(e_Wo)) + e_bo
    h1b, c1b = _cell(xw, _dot(h1a, w(e_Wh1)), e_b1, c1a)
    h2b, c2b = _cell(_dot(h1b, e_Wi2_v[...]), _dot(h2a, w(e_Wh2)), e_b2, c2a)
    out_b = _dot(h2b, e_Wo[...]) + e_bo
    e_h1, e_c1, e_h2, e_c2 = h1b, c1b, h2b, c2b

    o = w_lo * out_a + w_hi * out_b
    out_ref[:, 0, :] = o

    for t in range(1, T):
        xt = o
        d_h1, d_c1 = _cell(_dot(xt, d_Wi1_v[...]),
                           _dot(d_h1, w(d_Wh1)), d_b1, d_c1)
        d_h2, d_c2 = _cell(_dot(d_h1, d_Wi2_v[...]),
                           _dot(d_h2, w(d_Wh2)), d_b2, d_c2)
        w_lo, w_hi = gate_weights(d_c2)

        xw = _dot(xt, e_Wi1_v[...])
        h1a, c1a = _cell(xw, _dot(e_h1, e_Wh1[...]), e_b1, e_c1)
        h2a, c2a = _cell(_dot(h1a, e_Wi2_v[...]),
                         _dot(e_h2, e_Wh2[...]), e_b2, e_c2)
        out_a = _dot(h2a, e_Wo[...]) + e_bo
        h1b, c1b = _cell(xw, _dot(h1a, e_Wh1[...]), e_b1, c1a)
        h2b, c2b = _cell(_dot(h1b, e_Wi2_v[...]),
                         _dot(h2a, e_Wh2[...]), e_b2, c2a)
        out_b = _dot(h2b, e_Wo[...]) + e_bo
        e_h1, e_c1, e_h2, e_c2 = h1b, c1b, h2b, c2b

        o = w_lo * out_a + w_hi * out_b
        out_ref[:, t, :] = o


def kernel(x, d_Wi1, d_Wh1, d_b1, d_Wi2, d_Wh2, d_b2, d_Wo, d_bo,
           g_W, g_b,
           e_Wi1, e_Wh1, e_b1, e_Wi2, e_Wh2, e_b2, e_Wo, e_bo):
    del d_Wo, d_bo
    n_small = 8
    n_big = 5
    big_shapes = [(512, 2048)] * 4 + [(512, 512)]
    out = pl.pallas_call(
        _moe_kernel,
        out_shape=jax.ShapeDtypeStruct((B, T, H), jnp.float32),
        in_specs=(
            [pl.BlockSpec(memory_space=pltpu.MemorySpace.HBM)]
            + [pl.BlockSpec(memory_space=pltpu.MemorySpace.VMEM)] * (n_small - 1)
            + [pl.BlockSpec(memory_space=pltpu.MemorySpace.VMEM)] * 4
            + [pl.BlockSpec(memory_space=pltpu.MemorySpace.HBM)] * n_big),
        out_specs=pl.BlockSpec(memory_space=pltpu.MemorySpace.VMEM),
        scratch_shapes=(
            [pltpu.VMEM((B, H), jnp.float32)]
            + [pltpu.VMEM(s, jnp.float32) for s in big_shapes]
            + [pltpu.SemaphoreType.DMA] * (n_big + 1)),
    )(x, d_b1.reshape(1, -1), d_b2.reshape(1, -1), g_W, g_b.reshape(1, -1),
      e_b1.reshape(1, -1), e_b2.reshape(1, -1), e_bo.reshape(1, -1),
      d_Wi1, d_Wi2, e_Wi1, e_Wi2,
      d_Wh1, d_Wh2, e_Wh1, e_Wh2, e_Wo)
    return out
